# baseline (speedup 1.0000x reference)
import functools

import numpy as np
import jax
import jax.numpy as jnp
from jax import lax
from jax.experimental import pallas as pl
from jax.experimental.pallas import tpu as pltpu
from jax.experimental.pallas import tpu_sc as plsc

_BINS = np.array([-10, -2.15387469, -1.86273187, -1.67593972, -1.53412054,
                  -1.41779714, -1.3180109, -1.22985876, -1.15034938,
                  -1.07751557, -1.00999017, -0.94678176, -0.88714656,
                  -0.83051088, -0.77642176, -0.72451438, -0.67448975,
                  -0.62609901, -0.57913216, -0.53340971, -0.48877641,
                  -0.44509652, -0.40225007, -0.36012989, -0.31863936,
                  -0.27769044, -0.23720211, -0.19709908, -0.15731068,
                  -0.11776987, -0.07841241, -0.03917609, 0.0, 0.03917609,
                  0.07841241, 0.11776987, 0.15731068, 0.19709908, 0.23720211,
                  0.27769044, 0.31863936, 0.36012989, 0.40225007, 0.44509652,
                  0.48877641, 0.53340971, 0.57913216, 0.62609901, 0.67448975,
                  0.72451438, 0.77642176, 0.83051088, 0.88714656, 0.94678176,
                  1.00999017, 1.07751557, 1.15034938, 1.22985876, 1.3180109,
                  1.41779714, 1.53412054, 1.67593972, 1.86273187, 2.15387469,
                  10], dtype=np.float32)

_GLO = np.float32(-2.2)
_GHI = np.float32(2.2)
_NCELLS = 256
_INVH = np.float32(_NCELLS / (_GHI - _GLO))
_EDGES_HI = _BINS[1:65]
_cell_starts = _GLO + np.arange(_NCELLS, dtype=np.float32) / _INVH
_GUESS = np.searchsorted(_EDGES_HI, _cell_starts, side="left").astype(np.int32)
assert int(_GUESS.max()) <= 63

_CONST_F = np.stack([
    _BINS[1:65],
    _BINS[0:64],
    (1.0 / (_BINS[1:65] - _BINS[0:64] + 1e-8)).astype(np.float32),
]).astype(np.float32)

_L = 16
_NTILES = 32


def kernel(x, W, b):
    B, N = x.shape
    D = W.shape[1]
    ROWS = B // _NTILES
    NG = ROWS // _L

    xf = x.reshape(B * N)
    Wf = W.reshape(N, D * 64)
    constf = jnp.asarray(_CONST_F)
    guess = jnp.asarray(_GUESS)

    mesh = plsc.VectorSubcoreMesh(core_axis_name="c", subcore_axis_name="s")

    @functools.partial(
        pl.kernel,
        mesh=mesh,
        compiler_params=pltpu.CompilerParams(
            use_tc_tiling_on_sc=False, needs_layout_passes=False),
        out_type=jax.ShapeDtypeStruct((N, D // 8, B // 128, 8, 128),
                                      jnp.float32),
        scratch_types=[
            pltpu.VMEM((ROWS * N,), jnp.float32),
            pltpu.VMEM((D * 64,), jnp.float32),
            pltpu.VMEM((D * 64,), jnp.float32),
            pltpu.VMEM((D,), jnp.float32),
            pltpu.VMEM((D,), jnp.float32),
            pltpu.VMEM((64,), jnp.float32),
            pltpu.VMEM((64,), jnp.float32),
            pltpu.VMEM((64,), jnp.float32),
            pltpu.VMEM((_NCELLS,), jnp.int32),
            pltpu.VMEM((64 * D,), jnp.float32),
            pltpu.VMEM((64 * D,), jnp.float32),
            pltpu.VMEM((D // 8, ROWS // 128, 8, 128), jnp.float32),
            pltpu.VMEM((D // 8, ROWS // 128, 8, 128), jnp.float32),
            pltpu.SemaphoreType.DMA,
            pltpu.SemaphoreType.DMA,
            pltpu.SemaphoreType.DMA,
            pltpu.SemaphoreType.DMA,
            pltpu.SemaphoreType.DMA,
        ],
    )
    def run(x_h, W_h, b_h, cf_h, g_h, out_h,
            xblk, wb0, wb1, bb0, bb1, hi_e, lo_e, iw_e, gbuf, t0, t1,
            ob0, ob1, xsem, wsem0, wsem1, osem0, osem1):
        wid = lax.axis_index("c") * 16 + lax.axis_index("s")
        row0 = wid * ROWS
        bt0 = wid * (ROWS // 128)
        NBT = ROWS // 128
        pltpu.async_copy(x_h.at[pl.ds(row0 * N, ROWS * N)], xblk, xsem)
        pltpu.sync_copy(cf_h.at[0], hi_e)
        pltpu.sync_copy(cf_h.at[1], lo_e)
        pltpu.sync_copy(cf_h.at[2], iw_e)
        pltpu.sync_copy(g_h, gbuf)
        pltpu.async_copy(W_h.at[0], wb0, wsem0)
        pltpu.async_copy(b_h.at[0], bb0, wsem0)
        pltpu.async_copy(W_h.at[1], wb1, wsem1)
        pltpu.async_copy(b_h.at[1], bb1, wsem1)
        iota = lax.iota(jnp.int32, _L)
        iota64 = iota * 64
        iotaN = iota * N
        pltpu.make_async_copy(x_h.at[pl.ds(row0 * N, ROWS * N)], xblk,
                              xsem).wait()

        def one_field(n, i2, wb, bb, wsem, ob, osem):
            pltpu.make_async_copy(W_h.at[0], wb, wsem).wait()
            pltpu.make_async_copy(b_h.at[0], bb, wsem).wait()

            def build(j, acc):
                wrow = plsc.load_gather(wb, [iota64 + j])
                base = j * D + iota
                plsc.store_scatter(t1, [base], wrow)
                plsc.store_scatter(t0, [base], acc)
                return acc + wrow

            lax.fori_loop(0, 64, build, bb[...])

            nn = jnp.minimum(n + 2, N - 1)
            pltpu.async_copy(W_h.at[nn], wb, wsem)
            pltpu.async_copy(b_h.at[nn], bb, wsem)

            @pl.when(i2 >= 1)
            def _():
                pltpu.make_async_copy(
                    ob, out_h.at[0, :, pl.ds(bt0, NBT)], osem).wait()

            xoff = iotaN + n

            def group(g, c2):
                xv = plsc.load_gather(xblk, [g * (_L * N) + xoff])
                q = (jnp.clip(xv, _GLO, _GHI) - _GLO) * _INVH
                q = jnp.minimum(q.astype(jnp.int32), _NCELLS - 1)
                p0 = plsc.load_gather(gbuf, [q])
                edge = plsc.load_gather(hi_e, [p0])
                pos = jnp.minimum(p0 + jnp.where(edge < xv, 1, 0), 63)
                nv = (xv - plsc.load_gather(lo_e, [pos])) \
                    * plsc.load_gather(iw_e, [pos])
                tb = pos * D
                btl = lax.shift_right_logical(g, 3)
                c0 = lax.bitwise_and(g, 7) * _L
                for d in range(D):
                    g0 = plsc.load_gather(t0, [tb + d])
                    g1 = plsc.load_gather(t1, [tb + d])
                    r_ = jnp.maximum(g0 + nv * g1, 0.0)
                    ob[d // 8, btl, d % 8, pl.ds(c0, _L)] = r_
                return c2

            lax.fori_loop(0, NG, group, 0, unroll=2)
            pltpu.async_copy(ob, out_h.at[n, :, pl.ds(bt0, NBT)], osem)

        def pair(i2, carry):
            one_field(2 * i2, i2, wb0, bb0, wsem0, ob0, osem0)
            one_field(2 * i2 + 1, i2, wb1, bb1, wsem1, ob1, osem1)
            return carry

        lax.fori_loop(0, N // 2, pair, 0)
        pltpu.make_async_copy(ob0, out_h.at[0, :, pl.ds(bt0, NBT)],
                              osem0).wait()
        pltpu.make_async_copy(ob1, out_h.at[0, :, pl.ds(bt0, NBT)],
                              osem1).wait()
        pltpu.make_async_copy(W_h.at[0], wb0, wsem0).wait()
        pltpu.make_async_copy(b_h.at[0], bb0, wsem0).wait()
        pltpu.make_async_copy(W_h.at[0], wb1, wsem1).wait()
        pltpu.make_async_copy(b_h.at[0], bb1, wsem1).wait()

    res5 = run(xf, Wf, b, constf, guess)
    return jnp.transpose(res5, (2, 4, 0, 1, 3)).reshape(B, N, D)


# speedup vs baseline: 1.1573x; 1.1573x over previous
import functools

import numpy as np
import jax
import jax.numpy as jnp
from jax import lax
from jax.experimental import pallas as pl
from jax.experimental.pallas import tpu as pltpu
from jax.experimental.pallas import tpu_sc as plsc

_BINS = np.array([-10, -2.15387469, -1.86273187, -1.67593972, -1.53412054,
                  -1.41779714, -1.3180109, -1.22985876, -1.15034938,
                  -1.07751557, -1.00999017, -0.94678176, -0.88714656,
                  -0.83051088, -0.77642176, -0.72451438, -0.67448975,
                  -0.62609901, -0.57913216, -0.53340971, -0.48877641,
                  -0.44509652, -0.40225007, -0.36012989, -0.31863936,
                  -0.27769044, -0.23720211, -0.19709908, -0.15731068,
                  -0.11776987, -0.07841241, -0.03917609, 0.0, 0.03917609,
                  0.07841241, 0.11776987, 0.15731068, 0.19709908, 0.23720211,
                  0.27769044, 0.31863936, 0.36012989, 0.40225007, 0.44509652,
                  0.48877641, 0.53340971, 0.57913216, 0.62609901, 0.67448975,
                  0.72451438, 0.77642176, 0.83051088, 0.88714656, 0.94678176,
                  1.00999017, 1.07751557, 1.15034938, 1.22985876, 1.3180109,
                  1.41779714, 1.53412054, 1.67593972, 1.86273187, 2.15387469,
                  10], dtype=np.float32)

_GLO = np.float32(-2.2)
_GHI = np.float32(2.2)
_NCELLS = 256
_INVH = np.float32(_NCELLS / (_GHI - _GLO))
_EDGES_HI = _BINS[1:65]
_cell_starts = _GLO + np.arange(_NCELLS, dtype=np.float32) / _INVH
_GUESS = np.searchsorted(_EDGES_HI, _cell_starts, side="left").astype(np.int32)
assert int(_GUESS.max()) <= 63

_CONST_F = np.stack([
    _BINS[1:65],
    _BINS[0:64],
    (1.0 / (_BINS[1:65] - _BINS[0:64] + 1e-8)).astype(np.float32),
]).astype(np.float32)

_L = 16
_NTILES = 32


def kernel(x, W, b):
    B, N = x.shape
    D = W.shape[1]
    ROWS = B // _NTILES
    NG = ROWS // _L

    xf = x.reshape(B * N)
    Wf = W.reshape(N, D * 64)
    constf = jnp.asarray(_CONST_F)
    guess = jnp.asarray(_GUESS)

    mesh = plsc.VectorSubcoreMesh(core_axis_name="c", subcore_axis_name="s")

    @functools.partial(
        pl.kernel,
        mesh=mesh,
        compiler_params=pltpu.CompilerParams(
            use_tc_tiling_on_sc=False, needs_layout_passes=False),
        out_type=jax.ShapeDtypeStruct((N, D // 8, B // 128, 8, 128),
                                      jnp.float32),
        scratch_types=[
            pltpu.VMEM((ROWS * N,), jnp.float32),
            pltpu.VMEM((D * 64,), jnp.float32),
            pltpu.VMEM((D * 64,), jnp.float32),
            pltpu.VMEM((D,), jnp.float32),
            pltpu.VMEM((D,), jnp.float32),
            pltpu.VMEM((64,), jnp.float32),
            pltpu.VMEM((64,), jnp.float32),
            pltpu.VMEM((64,), jnp.float32),
            pltpu.VMEM((_NCELLS,), jnp.int32),
            pltpu.VMEM((64 * D,), jnp.float32),
            pltpu.VMEM((64 * D,), jnp.float32),
            pltpu.VMEM((D // 8, ROWS // 128, 8, 128), jnp.float32),
            pltpu.VMEM((D // 8, ROWS // 128, 8, 128), jnp.float32),
            pltpu.SemaphoreType.DMA,
            pltpu.SemaphoreType.DMA,
            pltpu.SemaphoreType.DMA,
            pltpu.SemaphoreType.DMA,
            pltpu.SemaphoreType.DMA,
        ],
    )
    def run(x_h, W_h, b_h, cf_h, g_h, out_h,
            xblk, wb0, wb1, bb0, bb1, hi_e, lo_e, iw_e, gbuf, t0, t1,
            ob0, ob1, xsem, wsem0, wsem1, osem0, osem1):
        wid = lax.axis_index("c") * 16 + lax.axis_index("s")
        row0 = wid * ROWS
        bt0 = wid * (ROWS // 128)
        NBT = ROWS // 128
        pltpu.async_copy(x_h.at[pl.ds(row0 * N, ROWS * N)], xblk, xsem)
        pltpu.sync_copy(cf_h.at[0], hi_e)
        pltpu.sync_copy(cf_h.at[1], lo_e)
        pltpu.sync_copy(cf_h.at[2], iw_e)
        pltpu.sync_copy(g_h, gbuf)
        pltpu.async_copy(W_h.at[0], wb0, wsem0)
        pltpu.async_copy(b_h.at[0], bb0, wsem0)
        pltpu.async_copy(W_h.at[1], wb1, wsem1)
        pltpu.async_copy(b_h.at[1], bb1, wsem1)
        iota = lax.iota(jnp.int32, _L)
        iota64 = iota * 64
        iotaN = iota * N
        dt_vec = lax.shift_right_logical(iota, 3)
        rr_vec = lax.bitwise_and(iota, 7)
        pltpu.make_async_copy(x_h.at[pl.ds(row0 * N, ROWS * N)], xblk,
                              xsem).wait()

        def one_field(n, i2, wb, bb, wsem, ob, osem):
            pltpu.make_async_copy(W_h.at[0], wb, wsem).wait()
            pltpu.make_async_copy(b_h.at[0], bb, wsem).wait()

            def build(j, acc):
                wrow = plsc.load_gather(wb, [iota64 + j])
                base = j * D + iota
                plsc.store_scatter(t1, [base], wrow)
                plsc.store_scatter(t0, [base], acc)
                return acc + wrow

            lax.fori_loop(0, 64, build, bb[...])

            nn = jnp.minimum(n + 2, N - 1)
            pltpu.async_copy(W_h.at[nn], wb, wsem)
            pltpu.async_copy(b_h.at[nn], bb, wsem)

            @pl.when(i2 >= 1)
            def _():
                pltpu.make_async_copy(
                    ob, out_h.at[0, :, pl.ds(bt0, NBT)], osem).wait()

            xoff = iotaN + n

            def group(g, c2):
                xv = plsc.load_gather(xblk, [g * (_L * N) + xoff])
                q = (jnp.clip(xv, _GLO, _GHI) - _GLO) * _INVH
                q = jnp.minimum(q.astype(jnp.int32), _NCELLS - 1)
                p0 = plsc.load_gather(gbuf, [q])
                edge = plsc.load_gather(hi_e, [p0])
                pos = jnp.minimum(p0 + jnp.where(edge < xv, 1, 0), 63)
                nv = (xv - plsc.load_gather(lo_e, [pos])) \
                    * plsc.load_gather(iw_e, [pos])
                pv = pos * D
                btv = jnp.full((_L,), lax.shift_right_logical(g, 3),
                               jnp.int32)
                cbase = jnp.full((_L,), lax.bitwise_and(g, 7) * _L,
                                 jnp.int32)
                for k in range(_L):
                    off = pv[k]
                    nm = nv[k]
                    r0 = t0[pl.ds(off, _L)]
                    r1 = t1[pl.ds(off, _L)]
                    plsc.store_scatter(ob, [dt_vec, btv, rr_vec, cbase + k],
                                       jnp.maximum(r0 + nm * r1, 0.0))
                return c2

            lax.fori_loop(0, NG, group, 0, unroll=2)
            pltpu.async_copy(ob, out_h.at[n, :, pl.ds(bt0, NBT)], osem)

        def pair(i2, carry):
            one_field(2 * i2, i2, wb0, bb0, wsem0, ob0, osem0)
            one_field(2 * i2 + 1, i2, wb1, bb1, wsem1, ob1, osem1)
            return carry

        lax.fori_loop(0, N // 2, pair, 0)
        pltpu.make_async_copy(ob0, out_h.at[0, :, pl.ds(bt0, NBT)],
                              osem0).wait()
        pltpu.make_async_copy(ob1, out_h.at[0, :, pl.ds(bt0, NBT)],
                              osem1).wait()
        pltpu.make_async_copy(W_h.at[0], wb0, wsem0).wait()
        pltpu.make_async_copy(b_h.at[0], bb0, wsem0).wait()
        pltpu.make_async_copy(W_h.at[0], wb1, wsem1).wait()
        pltpu.make_async_copy(b_h.at[0], bb1, wsem1).wait()

    res5 = run(xf, Wf, b, constf, guess)
    return jnp.transpose(res5, (2, 4, 0, 1, 3)).reshape(B, N, D)
